# speedup vs baseline: 2.8723x; 2.8723x over previous
import jax
import jax.numpy as jnp
from jax.experimental import pallas as pl

SIZE = 128
VEC = SIZE * (SIZE + 1) // 2
BLOCK_B = 8


def _body(vec_ref, out_ref):
    col = jax.lax.broadcasted_iota(jnp.int32, (1, SIZE), 1)
    for i in range(SIZE):
        tri = i * (i + 1) // 2
        row = vec_ref[:, tri:tri + SIZE]
        d = vec_ref[:, tri + i:tri + i + 1]
        sp = jnp.logaddexp(d, 0.0)
        masked = jnp.where(col < i, row, 0.0)
        out_ref[:, i, :] = jnp.where(col == i, sp, masked)


def kernel(L_vec):
    batch = L_vec.shape[0]
    return pl.pallas_call(
        _body,
        grid=(batch // BLOCK_B,),
        in_specs=[pl.BlockSpec((BLOCK_B, VEC), lambda b: (b, 0))],
        out_specs=pl.BlockSpec((BLOCK_B, SIZE, SIZE), lambda b: (b, 0, 0)),
        out_shape=jax.ShapeDtypeStruct((batch, SIZE, SIZE), jnp.float32),
    )(L_vec)
